# baseline (speedup 1.0000x reference)
import jax
import jax.numpy as jnp
from jax import lax
from jax.experimental import pallas as pl
from jax.experimental.pallas import tpu as pltpu
from jax.experimental.pallas import tpu_sc as plsc

_TOPK = 1024
_B = 128
_N = 32768
_L = 16
_NBINS = 256
_NW = 32
_ROWS_PER_W = _B // _NW
_NR = 256
_NK = 128 // _L

_INT_MIN = -2147483648
_CBUF = 12288


def _splat(s):
    return lax.broadcast_in_dim(jnp.int32(s), (_L,), ())


def _mono_u(xv):
    b = lax.bitcast_convert_type(xv, jnp.int32)
    m = lax.shift_right_arithmetic(b, _splat(31))
    return b ^ (m | _splat(_INT_MIN))


def _body(x_hbm, out_hbm, a0, a1, u_v, cbuf, h0, h1, h2, in_sem, out_sem):
    nc = 2
    wid = lax.axis_index("s") * nc + lax.axis_index("c")
    iota = lax.iota(jnp.int32, _L)
    lane_base = iota * _NBINS
    ones = jnp.full((_L,), 1, jnp.int32)
    zeros = jnp.full((_L,), 0, jnp.int32)
    bufs = (a0, a1)
    unroll = 1

    def in_copy(r):
        row = wid * _ROWS_PER_W + r
        return pltpu.async_copy(
            x_hbm.at[row // 8, :, row % 8, :], bufs[r % 2], in_sem)

    def out_copy(r):
        row = wid * _ROWS_PER_W + r
        return pltpu.async_copy(
            bufs[r % 2], out_hbm.at[row // 8, :, row % 8, :], out_sem)

    pend_out = [None] * _ROWS_PER_W
    h_in = in_copy(0)

    for r in range(_ROWS_PER_W):
        a = bufs[r % 2]

        @plsc.parallel_loop(0, _NBINS, unroll=4)
        def _(i):
            off = i * _L
            for h in (h0, h1, h2):
                h[pl.ds(off, _L)] = zeros

        h_in.wait()

        @plsc.parallel_loop(0, _NR, unroll=unroll)
        def _(i):
            for k in range(_NK):
                sl = pl.ds(k * _L, _L)
                u = _mono_u(a[i, sl])
                u_v[i, sl] = u
                bin_ = lax.shift_right_logical(u, _splat(24))
                plsc.addupdate_scatter(h0, [lane_base + bin_], ones)

        if r + 1 < _ROWS_PER_W:
            if r - 1 >= 0:
                pend_out[r - 1].wait()
            h_in = in_copy(r + 1)

        def level_scan(h, kr):
            def body(c_rev, carry):
                found, bstar, above, run = carry
                c = 15 - c_rev
                off = c * _L
                t = zeros
                for j in range(_L):
                    t = t + h[pl.ds(j * _NBINS + off, _L)]
                rv = lax.rev(t, (0,))
                cs = plsc.cumsum(rv)
                acc = run + cs
                crossed = acc >= kr
                npop = plsc.all_reduce_population_count(crossed)
                any_c = npop > 0
                j1 = plsc.all_reduce_ffs(crossed)
                sel = iota == j1
                a_at = jnp.sum(jnp.where(sel, acc, 0))
                t_at = jnp.sum(jnp.where(sel, rv, 0))
                bin_here = _splat(0) + (c * _L + 15 - j1)
                above_here = zeros + (a_at - t_at)
                take = jnp.logical_and(jnp.logical_not(found), any_c)
                bstar = jnp.where(take, bin_here, bstar)
                above = jnp.where(take, above_here, above)
                found = jnp.logical_or(found, any_c)
                run = run + (zeros + jnp.sum(t))
                return found, bstar, above, run
            init = (iota < 0, zeros, zeros, zeros)
            _, bstar, above, _ = lax.fori_loop(0, 16, body, init)
            return bstar, above

        kr = _splat(_TOPK)
        b1, above = level_scan(h0, kr)
        kr = kr - above

        cap = _CBUF - _L

        @plsc.parallel_loop(0, _NR, unroll=unroll, carry=zeros)
        def cnt(i, cnt):
            for k in range(_NK):
                u = u_v[i, pl.ds(k * _L, _L)]
                msk = lax.shift_right_logical(u, _splat(24)) == b1
                bin_ = lax.shift_right_logical(u, _splat(16)) & _splat(0xFF)
                plsc.addupdate_scatter(h1, [lane_base + bin_], ones, mask=msk)
                mi = jnp.where(msk, ones, zeros)
                pos = cnt + plsc.cumsum(mi) - mi
                msk2 = jnp.logical_and(msk, pos < _splat(cap))
                plsc.store_scatter(cbuf, [pos], u, mask=msk2)
                cnt = cnt + plsc.all_reduce_population_count(msk)
            return cnt

        m1 = jnp.minimum(jnp.max(cnt), jnp.int32(cap))
        pad_base = (m1 >> 4) << 4
        plsc.store_scatter(cbuf, [pad_base + iota], zeros,
                           mask=iota >= (m1 & 15))
        nv1 = (m1 + 15) >> 4

        b2, above = level_scan(h1, kr)
        kr = kr - above
        prefix2 = (b1 << _splat(8)) | b2

        @plsc.parallel_loop(0, nv1, unroll=4)
        def _(i):
            u = cbuf[pl.ds(i * _L, _L)]
            msk = lax.shift_right_logical(u, _splat(16)) == prefix2
            bin_ = lax.shift_right_logical(u, _splat(8)) & _splat(0xFF)
            plsc.addupdate_scatter(h2, [lane_base + bin_], ones, mask=msk)

        b3, above = level_scan(h2, kr)
        kr = kr - above
        prefix3 = (prefix2 << _splat(8)) | b3

        @plsc.parallel_loop(0, _NBINS, unroll=4)
        def _(i):
            h0[pl.ds(i * _L, _L)] = zeros

        @plsc.parallel_loop(0, nv1, unroll=4)
        def _(i):
            u = cbuf[pl.ds(i * _L, _L)]
            msk = lax.shift_right_logical(u, _splat(8)) == prefix3
            bin_ = u & _splat(0xFF)
            plsc.addupdate_scatter(h0, [lane_base + bin_], ones, mask=msk)

        b4, _above = level_scan(h0, kr)
        prefix = (prefix3 << _splat(8)) | b4

        thr = prefix ^ _splat(_INT_MIN)

        @plsc.parallel_loop(0, _NR, unroll=unroll)
        def _(i):
            for k in range(_NK):
                sl = pl.ds(k * _L, _L)
                w = u_v[i, sl] ^ _splat(_INT_MIN)
                keep = w >= thr
                m2 = lax.shift_right_arithmetic(w, _splat(31))
                b = w ^ lax.shift_right_logical(m2, _splat(1))
                xv = lax.bitcast_convert_type(b, jnp.float32)
                a[i, sl] = jnp.where(keep, xv, 0.0)

        pend_out[r] = out_copy(r)

    pend_out[_ROWS_PER_W - 2].wait()
    pend_out[_ROWS_PER_W - 1].wait()


@jax.jit
def kernel(x):
    mesh = plsc.VectorSubcoreMesh(core_axis_name="c", subcore_axis_name="s")
    fn = pl.kernel(
        _body,
        out_type=jax.ShapeDtypeStruct((_B // 8, _NR, 8, 128), jnp.float32),
        mesh=mesh,
        compiler_params=pltpu.CompilerParams(
            needs_layout_passes=False, disable_bounds_checks=True),
        scratch_types=[
            pltpu.VMEM((_NR, 128), jnp.float32),
            pltpu.VMEM((_NR, 128), jnp.float32),
            pltpu.VMEM((_NR, 128), jnp.int32),
            pltpu.VMEM((_CBUF,), jnp.int32),
            pltpu.VMEM((_L * _NBINS,), jnp.int32),
            pltpu.VMEM((_L * _NBINS,), jnp.int32),
            pltpu.VMEM((_L * _NBINS,), jnp.int32),
            pltpu.SemaphoreType.DMA,
            pltpu.SemaphoreType.DMA,
        ],
    )
    xt = x.reshape(_B // 8, 8, _NR, 128).transpose(0, 2, 1, 3)
    out = fn(xt)
    return out.transpose(0, 2, 1, 3).reshape(_B, _N)


# speedup vs baseline: 1.1154x; 1.1154x over previous
import jax
import jax.numpy as jnp
from jax import lax
from jax.experimental import pallas as pl
from jax.experimental.pallas import tpu as pltpu
from jax.experimental.pallas import tpu_sc as plsc

_TOPK = 1024
_B = 128
_N = 32768
_L = 16
_NBINS = 256
_NW = 32
_ROWS_PER_W = _B // _NW
_NR = 256
_NK = 128 // _L

_INT_MIN = -2147483648
_CLANE = 768
_CBUF = _L * _CLANE


def _splat(s):
    return lax.broadcast_in_dim(jnp.int32(s), (_L,), ())


def _mono_u(xv):
    b = lax.bitcast_convert_type(xv, jnp.int32)
    m = lax.shift_right_arithmetic(b, _splat(31))
    return b ^ (m | _splat(_INT_MIN))


def _body(x_hbm, out_hbm, a0, a1, u_v, cbuf, h0, h1, h2, in_sem, out_sem):
    nc = 2
    wid = lax.axis_index("s") * nc + lax.axis_index("c")
    iota = lax.iota(jnp.int32, _L)
    lane_base = iota * _NBINS
    ones = jnp.full((_L,), 1, jnp.int32)
    zeros = jnp.full((_L,), 0, jnp.int32)
    bufs = (a0, a1)
    unroll = 1

    def in_copy(r):
        row = wid * _ROWS_PER_W + r
        return pltpu.async_copy(
            x_hbm.at[row // 8, :, row % 8, :], bufs[r % 2], in_sem)

    def out_copy(r):
        row = wid * _ROWS_PER_W + r
        return pltpu.async_copy(
            bufs[r % 2], out_hbm.at[row // 8, :, row % 8, :], out_sem)

    pend_out = [None] * _ROWS_PER_W
    h_in = in_copy(0)

    for r in range(_ROWS_PER_W):
        a = bufs[r % 2]

        @plsc.parallel_loop(0, _NBINS, unroll=4)
        def _(i):
            off = i * _L
            for h in (h0, h1, h2):
                h[pl.ds(off, _L)] = zeros

        h_in.wait()

        @plsc.parallel_loop(0, _NR, unroll=unroll)
        def _(i):
            for k in range(_NK):
                sl = pl.ds(k * _L, _L)
                u = _mono_u(a[i, sl])
                u_v[i, sl] = u
                bin_ = lax.shift_right_logical(u, _splat(24))
                plsc.addupdate_scatter(h0, [lane_base + bin_], ones)

        if r + 1 < _ROWS_PER_W:
            if r - 1 >= 0:
                pend_out[r - 1].wait()
            h_in = in_copy(r + 1)

        def level_scan(h, kr):
            def body(c_rev, carry):
                found, bstar, above, run = carry
                c = 15 - c_rev
                off = c * _L
                t = zeros
                for j in range(_L):
                    t = t + h[pl.ds(j * _NBINS + off, _L)]
                rv = lax.rev(t, (0,))
                cs = plsc.cumsum(rv)
                acc = run + cs
                crossed = acc >= kr
                npop = plsc.all_reduce_population_count(crossed)
                any_c = npop > 0
                j1 = plsc.all_reduce_ffs(crossed)
                sel = iota == j1
                a_at = jnp.sum(jnp.where(sel, acc, 0))
                t_at = jnp.sum(jnp.where(sel, rv, 0))
                bin_here = _splat(0) + (c * _L + 15 - j1)
                above_here = zeros + (a_at - t_at)
                take = jnp.logical_and(jnp.logical_not(found), any_c)
                bstar = jnp.where(take, bin_here, bstar)
                above = jnp.where(take, above_here, above)
                found = jnp.logical_or(found, any_c)
                run = run + (zeros + jnp.sum(t))
                return found, bstar, above, run
            init = (iota < 0, zeros, zeros, zeros)
            _, bstar, above, _ = lax.fori_loop(0, 16, body, init)
            return bstar, above

        kr = _splat(_TOPK)
        b1, above = level_scan(h0, kr)
        kr = kr - above

        bucket_base = iota * _CLANE

        @plsc.parallel_loop(0, _NR, unroll=unroll, carry=zeros)
        def cnt(i, cnt):
            for k in range(_NK):
                u = u_v[i, pl.ds(k * _L, _L)]
                msk = lax.shift_right_logical(u, _splat(24)) == b1
                bin_ = lax.shift_right_logical(u, _splat(16)) & _splat(0xFF)
                plsc.addupdate_scatter(h1, [lane_base + bin_], ones, mask=msk)
                ccnt = jnp.minimum(cnt, _splat(_CLANE - 1))
                plsc.store_scatter(cbuf, [bucket_base + ccnt], u, mask=msk)
                cnt = cnt + jnp.where(msk, ones, zeros)
            return cnt

        nv1 = jnp.max(cnt)

        b2, above = level_scan(h1, kr)
        kr = kr - above
        prefix2 = (b1 << _splat(8)) | b2

        @plsc.parallel_loop(0, nv1, unroll=4)
        def _(i):
            u = plsc.load_gather(cbuf, [bucket_base + i])
            msk = jnp.logical_and(
                lax.shift_right_logical(u, _splat(16)) == prefix2,
                _splat(0) + i < cnt)
            bin_ = lax.shift_right_logical(u, _splat(8)) & _splat(0xFF)
            plsc.addupdate_scatter(h2, [lane_base + bin_], ones, mask=msk)

        b3, above = level_scan(h2, kr)
        kr = kr - above
        prefix3 = (prefix2 << _splat(8)) | b3

        @plsc.parallel_loop(0, _NBINS, unroll=4)
        def _(i):
            h0[pl.ds(i * _L, _L)] = zeros

        @plsc.parallel_loop(0, nv1, unroll=4)
        def _(i):
            u = plsc.load_gather(cbuf, [bucket_base + i])
            msk = jnp.logical_and(
                lax.shift_right_logical(u, _splat(8)) == prefix3,
                _splat(0) + i < cnt)
            bin_ = u & _splat(0xFF)
            plsc.addupdate_scatter(h0, [lane_base + bin_], ones, mask=msk)

        b4, _above = level_scan(h0, kr)
        prefix = (prefix3 << _splat(8)) | b4

        thr = prefix ^ _splat(_INT_MIN)

        @plsc.parallel_loop(0, _NR, unroll=unroll)
        def _(i):
            for k in range(_NK):
                sl = pl.ds(k * _L, _L)
                w = u_v[i, sl] ^ _splat(_INT_MIN)
                keep = w >= thr
                m2 = lax.shift_right_arithmetic(w, _splat(31))
                b = w ^ lax.shift_right_logical(m2, _splat(1))
                xv = lax.bitcast_convert_type(b, jnp.float32)
                a[i, sl] = jnp.where(keep, xv, 0.0)

        pend_out[r] = out_copy(r)

    pend_out[_ROWS_PER_W - 2].wait()
    pend_out[_ROWS_PER_W - 1].wait()


@jax.jit
def kernel(x):
    mesh = plsc.VectorSubcoreMesh(core_axis_name="c", subcore_axis_name="s")
    fn = pl.kernel(
        _body,
        out_type=jax.ShapeDtypeStruct((_B // 8, _NR, 8, 128), jnp.float32),
        mesh=mesh,
        compiler_params=pltpu.CompilerParams(
            needs_layout_passes=False, disable_bounds_checks=True),
        scratch_types=[
            pltpu.VMEM((_NR, 128), jnp.float32),
            pltpu.VMEM((_NR, 128), jnp.float32),
            pltpu.VMEM((_NR, 128), jnp.int32),
            pltpu.VMEM((_CBUF,), jnp.int32),
            pltpu.VMEM((_L * _NBINS,), jnp.int32),
            pltpu.VMEM((_L * _NBINS,), jnp.int32),
            pltpu.VMEM((_L * _NBINS,), jnp.int32),
            pltpu.SemaphoreType.DMA,
            pltpu.SemaphoreType.DMA,
        ],
    )
    xt = x.reshape(_B // 8, 8, _NR, 128).transpose(0, 2, 1, 3)
    out = fn(xt)
    return out.transpose(0, 2, 1, 3).reshape(_B, _N)


# speedup vs baseline: 1.9190x; 1.7206x over previous
import jax
import jax.numpy as jnp
from jax import lax
from jax.experimental import pallas as pl
from jax.experimental.pallas import tpu as pltpu
from jax.experimental.pallas import tpu_sc as plsc

_TOPK = 1024
_B = 128
_N = 32768
_L = 16
_NBINS = 256
_NW = 32
_ROWS_PER_W = _B // _NW
_NR = 256
_NK = 128 // _L

_INT_MIN = -2147483648


def _splat(s):
    return lax.broadcast_in_dim(jnp.int32(s), (_L,), ())


def _mono_u(xv):
    b = lax.bitcast_convert_type(xv, jnp.int32)
    m = lax.shift_right_arithmetic(b, _splat(31))
    return b ^ (m | _splat(_INT_MIN))


def _body(x_hbm, out_hbm, a0, a1, u_v, h0, h1, h2, in_sem, out_sem):
    nc = 2
    wid = lax.axis_index("s") * nc + lax.axis_index("c")
    iota = lax.iota(jnp.int32, _L)
    lane_base = iota * _NBINS
    ones = jnp.full((_L,), 1, jnp.int32)
    zeros = jnp.full((_L,), 0, jnp.int32)
    bufs = (a0, a1)
    unroll = 1

    def in_copy(r):
        row = wid * _ROWS_PER_W + r
        return pltpu.async_copy(
            x_hbm.at[row // 8, :, row % 8, :], bufs[r % 2], in_sem)

    def out_copy(r):
        row = wid * _ROWS_PER_W + r
        return pltpu.async_copy(
            bufs[r % 2], out_hbm.at[row // 8, :, row % 8, :], out_sem)

    pend_out = [None] * _ROWS_PER_W
    h_in = in_copy(0)

    for r in range(_ROWS_PER_W):
        a = bufs[r % 2]

        @plsc.parallel_loop(0, _NBINS // _L, unroll=4)
        def _(i):
            off = i * _L
            for h in (h0, h1, h2):
                h[pl.ds(off, _L)] = zeros

        h_in.wait()

        @plsc.parallel_loop(0, _NR, unroll=unroll)
        def _(i):
            for k in range(_NK):
                sl = pl.ds(k * _L, _L)
                u = _mono_u(a[i, sl])
                u_v[i, sl] = u
                bin_ = lax.shift_right_logical(u, _splat(24))
                plsc.addupdate_scatter(h0, [bin_], ones)

        if r + 1 < _ROWS_PER_W:
            if r - 1 >= 0:
                pend_out[r - 1].wait()
            h_in = in_copy(r + 1)

        def hist_pass(h, shift, prefix):
            @plsc.parallel_loop(0, _NR, unroll=unroll)
            def _(i):
                for k in range(_NK):
                    u = u_v[i, pl.ds(k * _L, _L)]
                    bin_ = lax.shift_right_logical(u, _splat(shift)) & _splat(0xFF)
                    msk = lax.shift_right_logical(u, _splat(shift + 8)) == prefix
                    plsc.addupdate_scatter(h, [bin_], ones, mask=msk)

        def level_scan(h, kr):
            def body(c_rev, carry):
                found, bstar, above, run = carry
                c = 15 - c_rev
                off = c * _L
                t = h[pl.ds(off, _L)]
                rv = lax.rev(t, (0,))
                cs = plsc.cumsum(rv)
                acc = run + cs
                crossed = acc >= kr
                npop = plsc.all_reduce_population_count(crossed)
                any_c = npop > 0
                j1 = plsc.all_reduce_ffs(crossed)
                sel = iota == j1
                a_at = jnp.sum(jnp.where(sel, acc, 0))
                t_at = jnp.sum(jnp.where(sel, rv, 0))
                bin_here = _splat(0) + (c * _L + 15 - j1)
                above_here = zeros + (a_at - t_at)
                take = jnp.logical_and(jnp.logical_not(found), any_c)
                bstar = jnp.where(take, bin_here, bstar)
                above = jnp.where(take, above_here, above)
                found = jnp.logical_or(found, any_c)
                run = run + (zeros + jnp.sum(t))
                return found, bstar, above, run
            init = (iota < 0, zeros, zeros, zeros)
            _, bstar, above, _ = lax.fori_loop(0, 16, body, init)
            return bstar, above

        kr = _splat(_TOPK)
        prefix = None
        hist_for_lvl = (h0, h1, h2, h0)
        for lvl in range(4):
            shift = 24 - 8 * lvl
            if lvl == 3:
                @plsc.parallel_loop(0, _NBINS // _L, unroll=4)
                def _(i):
                    h0[pl.ds(i * _L, _L)] = zeros
            if lvl > 0:
                hist_pass(hist_for_lvl[lvl], shift, prefix)
            bstar, above = level_scan(hist_for_lvl[lvl], kr)
            kr = kr - above
            prefix = bstar if prefix is None else ((prefix << _splat(8)) | bstar)

        thr = prefix ^ _splat(_INT_MIN)

        @plsc.parallel_loop(0, _NR, unroll=unroll)
        def _(i):
            for k in range(_NK):
                sl = pl.ds(k * _L, _L)
                w = u_v[i, sl] ^ _splat(_INT_MIN)
                keep = w >= thr
                m2 = lax.shift_right_arithmetic(w, _splat(31))
                b = w ^ lax.shift_right_logical(m2, _splat(1))
                xv = lax.bitcast_convert_type(b, jnp.float32)
                a[i, sl] = jnp.where(keep, xv, 0.0)

        pend_out[r] = out_copy(r)

    pend_out[_ROWS_PER_W - 2].wait()
    pend_out[_ROWS_PER_W - 1].wait()


@jax.jit
def kernel(x):
    mesh = plsc.VectorSubcoreMesh(core_axis_name="c", subcore_axis_name="s")
    fn = pl.kernel(
        _body,
        out_type=jax.ShapeDtypeStruct((_B // 8, _NR, 8, 128), jnp.float32),
        mesh=mesh,
        compiler_params=pltpu.CompilerParams(
            needs_layout_passes=False, disable_bounds_checks=True),
        scratch_types=[
            pltpu.VMEM((_NR, 128), jnp.float32),
            pltpu.VMEM((_NR, 128), jnp.float32),
            pltpu.VMEM((_NR, 128), jnp.int32),
            pltpu.VMEM((_NBINS,), jnp.int32),
            pltpu.VMEM((_NBINS,), jnp.int32),
            pltpu.VMEM((_NBINS,), jnp.int32),
            pltpu.SemaphoreType.DMA,
            pltpu.SemaphoreType.DMA,
        ],
    )
    xt = x.reshape(_B // 8, 8, _NR, 128).transpose(0, 2, 1, 3)
    out = fn(xt)
    return out.transpose(0, 2, 1, 3).reshape(_B, _N)


# speedup vs baseline: 1.9282x; 1.0048x over previous
import jax
import jax.numpy as jnp
from jax import lax
from jax.experimental import pallas as pl
from jax.experimental.pallas import tpu as pltpu
from jax.experimental.pallas import tpu_sc as plsc

_TOPK = 1024
_B = 128
_N = 32768
_L = 16
_NBINS = 256
_NW = 32
_ROWS_PER_W = _B // _NW
_NR = 256
_NK = 128 // _L

_INT_MIN = -2147483648


def _splat(s):
    return lax.broadcast_in_dim(jnp.int32(s), (_L,), ())


def _mono_u(xv):
    b = lax.bitcast_convert_type(xv, jnp.int32)
    m = lax.shift_right_arithmetic(b, _splat(31))
    return b ^ (m | _splat(_INT_MIN))


def _body(x_hbm, out_hbm, a0, a1, u_v, h0, h1, h2, in_sem, out_sem):
    nc = 2
    wid = lax.axis_index("s") * nc + lax.axis_index("c")
    iota = lax.iota(jnp.int32, _L)
    lane_base = iota * _NBINS
    ones = jnp.full((_L,), 1, jnp.int32)
    zeros = jnp.full((_L,), 0, jnp.int32)
    bufs = (a0, a1)
    unroll = 1

    nch = 4
    chs = _NR // nch

    def in_copy(r, c=None):
        row = wid * _ROWS_PER_W + r
        if c is None:
            return pltpu.async_copy(
                x_hbm.at[row // 8, :, row % 8, :], bufs[r % 2], in_sem)
        sl = pl.ds(c * chs, chs)
        return pltpu.async_copy(
            x_hbm.at[row // 8, sl, row % 8, :],
            bufs[r % 2].at[sl, :], in_sem)

    def out_copy(r, c=None):
        row = wid * _ROWS_PER_W + r
        if c is None:
            return pltpu.async_copy(
                bufs[r % 2], out_hbm.at[row // 8, :, row % 8, :], out_sem)
        sl = pl.ds(c * chs, chs)
        return pltpu.async_copy(
            bufs[r % 2].at[sl, :],
            out_hbm.at[row // 8, sl, row % 8, :], out_sem)

    pend_out = [()] * _ROWS_PER_W
    h_in = [in_copy(0, c) for c in range(nch)]

    for r in range(_ROWS_PER_W):
        a = bufs[r % 2]

        @plsc.parallel_loop(0, _NBINS // _L, unroll=4)
        def _(i):
            off = i * _L
            for h in (h0, h1, h2):
                h[pl.ds(off, _L)] = zeros

        def l1_range(lo, hi):
            @plsc.parallel_loop(lo, hi, unroll=unroll)
            def _(i):
                for k in range(_NK):
                    sl = pl.ds(k * _L, _L)
                    u = _mono_u(a[i, sl])
                    u_v[i, sl] = u
                    bin_ = lax.shift_right_logical(u, _splat(24))
                    plsc.addupdate_scatter(h0, [bin_], ones)

        if len(h_in) > 1:
            for c in range(nch):
                h_in[c].wait()
                l1_range(c * chs, (c + 1) * chs)
        else:
            h_in[0].wait()
            l1_range(0, _NR)

        def hist_pass(h, shift, prefix):
            @plsc.parallel_loop(0, _NR, unroll=unroll)
            def _(i):
                for k in range(_NK):
                    u = u_v[i, pl.ds(k * _L, _L)]
                    bin_ = lax.shift_right_logical(u, _splat(shift)) & _splat(0xFF)
                    msk = lax.shift_right_logical(u, _splat(shift + 8)) == prefix
                    plsc.addupdate_scatter(h, [bin_], ones, mask=msk)

        def level_scan(h, kr):
            def body(c_rev, carry):
                found, bstar, above, run = carry
                c = 15 - c_rev
                off = c * _L
                t = h[pl.ds(off, _L)]
                rv = lax.rev(t, (0,))
                cs = plsc.cumsum(rv)
                acc = run + cs
                crossed = acc >= kr
                npop = plsc.all_reduce_population_count(crossed)
                any_c = npop > 0
                j1 = plsc.all_reduce_ffs(crossed)
                sel = iota == j1
                a_at = jnp.sum(jnp.where(sel, acc, 0))
                t_at = jnp.sum(jnp.where(sel, rv, 0))
                bin_here = _splat(0) + (c * _L + 15 - j1)
                above_here = zeros + (a_at - t_at)
                take = jnp.logical_and(jnp.logical_not(found), any_c)
                bstar = jnp.where(take, bin_here, bstar)
                above = jnp.where(take, above_here, above)
                found = jnp.logical_or(found, any_c)
                run = run + (zeros + jnp.sum(t))
                return found, bstar, above, run
            init = (iota < 0, zeros, zeros, zeros)
            _, bstar, above, _ = lax.fori_loop(0, 16, body, init)
            return bstar, above

        kr = _splat(_TOPK)
        prefix = None
        hist_for_lvl = (h0, h1, h2, h0)
        for lvl in range(4):
            shift = 24 - 8 * lvl
            if lvl == 3:
                @plsc.parallel_loop(0, _NBINS // _L, unroll=4)
                def _(i):
                    h0[pl.ds(i * _L, _L)] = zeros
            if lvl > 0:
                hist_pass(hist_for_lvl[lvl], shift, prefix)
            if lvl == 1 and r + 1 < _ROWS_PER_W:
                for h in pend_out[r - 1] if r >= 1 else ():
                    h.wait()
                h_in = [in_copy(r + 1)]
            bstar, above = level_scan(hist_for_lvl[lvl], kr)
            kr = kr - above
            prefix = bstar if prefix is None else ((prefix << _splat(8)) | bstar)

        thr = prefix ^ _splat(_INT_MIN)

        def out_range(lo, hi):
            @plsc.parallel_loop(lo, hi, unroll=unroll)
            def _(i):
                for k in range(_NK):
                    sl = pl.ds(k * _L, _L)
                    w = u_v[i, sl] ^ _splat(_INT_MIN)
                    keep = w >= thr
                    m2 = lax.shift_right_arithmetic(w, _splat(31))
                    b = w ^ lax.shift_right_logical(m2, _splat(1))
                    xv = lax.bitcast_convert_type(b, jnp.float32)
                    a[i, sl] = jnp.where(keep, xv, 0.0)

        if r == _ROWS_PER_W - 1:
            handles = []
            for c in range(nch):
                out_range(c * chs, (c + 1) * chs)
                handles.append(out_copy(r, c))
            pend_out[r] = tuple(handles)
        else:
            out_range(0, _NR)
            pend_out[r] = (out_copy(r),)

    for h in pend_out[_ROWS_PER_W - 2] + pend_out[_ROWS_PER_W - 1]:
        h.wait()


@jax.jit
def kernel(x):
    mesh = plsc.VectorSubcoreMesh(core_axis_name="c", subcore_axis_name="s")
    fn = pl.kernel(
        _body,
        out_type=jax.ShapeDtypeStruct((_B // 8, _NR, 8, 128), jnp.float32),
        mesh=mesh,
        compiler_params=pltpu.CompilerParams(
            needs_layout_passes=False, disable_bounds_checks=True),
        scratch_types=[
            pltpu.VMEM((_NR, 128), jnp.float32),
            pltpu.VMEM((_NR, 128), jnp.float32),
            pltpu.VMEM((_NR, 128), jnp.int32),
            pltpu.VMEM((_NBINS,), jnp.int32),
            pltpu.VMEM((_NBINS,), jnp.int32),
            pltpu.VMEM((_NBINS,), jnp.int32),
            pltpu.SemaphoreType.DMA,
            pltpu.SemaphoreType.DMA,
        ],
    )
    xt = x.reshape(_B // 8, 8, _NR, 128).transpose(0, 2, 1, 3)
    out = fn(xt)
    return out.transpose(0, 2, 1, 3).reshape(_B, _N)
